# Initial kernel scaffold; baseline (speedup 1.0000x reference)
import jax
import jax.numpy as jnp
from jax.experimental import pallas as pl


def kernel(features_0, w1_hm, b1_hm, w2_hm, b2_hm, w1_wh, b1_wh, w2_wh, b2_wh, w1_reg, b1_reg, w2_reg, b2_reg):
    raise NotImplementedError("write your pallas kernel here")


# speedup vs baseline: 1.0868x; 1.0868x over previous
import functools

import jax
import jax.numpy as jnp
from jax.experimental import pallas as pl
from jax.experimental.pallas import tpu as pltpu

_K = 100
_IMAGE_SIZE = 512


def _sigmoid(x):
    return 1.0 / (1.0 + jnp.exp(-x))


def _conv_body(x_ref, w1_ref, b1_ref, w2_ref, b2_ref, hm_ref, whreg_ref,
               acc_ref, *, H, W, NC):
    dy = pl.program_id(2)
    x = x_ref[0, 0]

    p = None
    for dx in range(3):
        xs = x[dx:dx + W, :]
        wk = w1_ref[pl.ds(3 * dy + dx, 1)][0]
        c = jax.lax.dot_general(xs, wk, (((1,), (0,)), ((), ())),
                                preferred_element_type=jnp.float32)
        p = c if p is None else p + c

    @pl.when(dy == 0)
    def _():
        acc_ref[:] = p

    @pl.when(dy > 0)
    def _():
        acc_ref[:] = acc_ref[:] + p

    @pl.when(dy == 2)
    def _():
        h = jnp.maximum(acc_ref[:] + b1_ref[:], 0.0)
        outT = jax.lax.dot_general(w2_ref[:], h, (((0,), (1,)), ((), ())),
                                   preferred_element_type=jnp.float32)
        outT = outT + b2_ref[:]
        is_hm = jax.lax.broadcasted_iota(jnp.int32, outT.shape, 0) < NC
        outT = jnp.where(is_hm, _sigmoid(outT), outT)
        hm_ref[:, 0, :] = outT[:NC, :]
        whreg_ref[0, :, :] = outT[NC:NC + 4, :]


def _nms_body(hm_ref, sc_ref, m_ref, *, H, W):
    s = hm_ref[0, 0].reshape(H, W)
    zc = jnp.zeros((H, 1), jnp.float32)
    h3 = jnp.maximum(s, jnp.concatenate([s[:, 1:], zc], axis=1))
    h3 = jnp.maximum(h3, jnp.concatenate([zc, s[:, :-1]], axis=1))
    zr = jnp.zeros((1, W), jnp.float32)
    v3 = jnp.maximum(h3, jnp.concatenate([h3[1:, :], zr], axis=0))
    v3 = jnp.maximum(v3, jnp.concatenate([zr, h3[:-1, :]], axis=0))
    sc = jnp.where(v3 == s, s, 0.0)
    sc_ref[0, 0] = sc.reshape(H * W)
    m_ref[0, 0, :] = jnp.max(sc, axis=1)


def _topk_body(sc_ref, m_ref, whreg_ref, det_ref, m2_ref, *, B, NC, H, W,
               K, down):
    m_all = m_ref[:].reshape(B, NC, H)
    m2_ref[:] = jnp.max(m_all, axis=2)

    iota_c = jax.lax.broadcasted_iota(jnp.int32, (1, NC), 1).reshape(NC)
    iota_w = jax.lax.broadcasted_iota(jnp.int32, (1, W), 1).reshape(W)
    big = jnp.int32(2 ** 30)

    def body(k, carry):
        for b in range(B):
            m2row = m2_ref[b, :]
            g = jnp.max(m2row)
            cstar = jnp.min(jnp.where(m2row == g, iota_c, big))
            mp = b * NC + cstar
            mrow = m_ref[pl.ds(mp, 1), 0, :].reshape(H)
            ystar = jnp.min(jnp.where(mrow == g, iota_w, big))
            srow = sc_ref[pl.ds(mp, 1), 0, pl.ds(ystar * W, W)].reshape(W)
            xstar = jnp.min(jnp.where(srow == g, iota_w, big))
            nrow = jnp.where(iota_w == xstar, -1.0, srow)
            sc_ref[pl.ds(mp, 1), 0, pl.ds(ystar * W, W)] = nrow.reshape(1, W)
            mrow2 = jnp.where(iota_w == ystar, jnp.max(nrow), mrow)
            m_ref[pl.ds(mp, 1), 0, :] = mrow2.reshape(1, H)
            m2_ref[b, :] = jnp.where(iota_c == cstar, jnp.max(mrow2), m2row)
            vals = []
            for ch in range(4):
                wrow = whreg_ref[b, ch, pl.ds(ystar * W, W)]
                vals.append(jnp.sum(jnp.where(iota_w == xstar, wrow, 0.0)))
            ww, hh, rx, ry = vals
            xf = xstar.astype(jnp.float32) + rx
            yf = ystar.astype(jnp.float32) + ry
            x1 = (xf - ww * 0.5) * down
            y1 = (yf - hh * 0.5) * down
            x2 = (xf + ww * 0.5) * down
            y2 = (yf + hh * 0.5) * down
            row = jnp.stack([x1, y1, x2, y2, g, cstar.astype(jnp.float32),
                             jnp.float32(0.0), jnp.float32(0.0)])
            det_ref[b, pl.ds(k, 1), :] = row.reshape(1, 8)
        return carry

    jax.lax.fori_loop(0, K, body, 0)


def kernel(features_0, w1_hm, b1_hm, w2_hm, b2_hm, w1_wh, b1_wh, w2_wh,
           b2_wh, w1_reg, b1_reg, w2_reg, b2_reg):
    B, C, H, W = features_0.shape
    HC = w1_hm.shape[0]
    NC = w2_hm.shape[0]
    HC3 = 3 * HC
    OC = 128
    W_pad = W + 8
    down = float(_IMAGE_SIZE) / W

    xt = jnp.transpose(features_0, (0, 2, 3, 1))
    xpad = jnp.pad(xt, ((0, 0), (1, 1), (1, 7), (0, 0)))

    def prep_w1(w):
        return jnp.transpose(w, (2, 3, 1, 0))

    w1cat = jnp.concatenate([prep_w1(w1_hm), prep_w1(w1_wh),
                             prep_w1(w1_reg)], axis=-1).reshape(9, C, HC3)
    b1cat = jnp.concatenate([b1_hm, b1_wh, b1_reg]).reshape(1, HC3)
    w2blk = jnp.zeros((HC3, OC), jnp.float32)
    w2blk = w2blk.at[0:HC, 0:NC].set(w2_hm.reshape(NC, HC).T)
    w2blk = w2blk.at[HC:2 * HC, NC:NC + 2].set(w2_wh.reshape(2, HC).T)
    w2blk = w2blk.at[2 * HC:3 * HC, NC + 2:NC + 4].set(w2_reg.reshape(2, HC).T)
    b2col = jnp.zeros((OC,), jnp.float32)
    b2col = b2col.at[0:NC].set(b2_hm)
    b2col = b2col.at[NC:NC + 2].set(b2_wh)
    b2col = b2col.at[NC + 2:NC + 4].set(b2_reg)
    b2col = b2col.reshape(OC, 1)

    hm, whreg = pl.pallas_call(
        functools.partial(_conv_body, H=H, W=W, NC=NC),
        grid=(B, H, 3),
        in_specs=[
            pl.BlockSpec((1, 1, W_pad, C), lambda b, y, dy: (b, y + dy, 0, 0)),
            pl.BlockSpec((9, C, HC3), lambda b, y, dy: (0, 0, 0)),
            pl.BlockSpec((1, HC3), lambda b, y, dy: (0, 0)),
            pl.BlockSpec((HC3, OC), lambda b, y, dy: (0, 0)),
            pl.BlockSpec((OC, 1), lambda b, y, dy: (0, 0)),
        ],
        out_specs=[
            pl.BlockSpec((NC, 1, W), lambda b, y, dy: (b, 0, y)),
            pl.BlockSpec((1, 4, W), lambda b, y, dy: (b, 0, y)),
        ],
        out_shape=[
            jax.ShapeDtypeStruct((B * NC, 1, H * W), jnp.float32),
            jax.ShapeDtypeStruct((B, 4, H * W), jnp.float32),
        ],
        scratch_shapes=[pltpu.VMEM((W, HC3), jnp.float32)],
        compiler_params=pltpu.CompilerParams(
            dimension_semantics=("parallel", "parallel", "arbitrary")),
    )(xpad, w1cat, b1cat, w2blk, b2col)

    scores, rowmax = pl.pallas_call(
        functools.partial(_nms_body, H=H, W=W),
        grid=(B * NC,),
        in_specs=[pl.BlockSpec((1, 1, H * W), lambda i: (i, 0, 0))],
        out_specs=[
            pl.BlockSpec((1, 1, H * W), lambda i: (i, 0, 0)),
            pl.BlockSpec((1, 1, H), lambda i: (i, 0, 0)),
        ],
        out_shape=[
            jax.ShapeDtypeStruct((B * NC, 1, H * W), jnp.float32),
            jax.ShapeDtypeStruct((B * NC, 1, H), jnp.float32),
        ],
        compiler_params=pltpu.CompilerParams(
            dimension_semantics=("arbitrary",)),
    )(hm)

    det = pl.pallas_call(
        functools.partial(_topk_body, B=B, NC=NC, H=H, W=W, K=_K, down=down),
        grid=(1,),
        in_specs=[
            pl.BlockSpec((B * NC, 1, H * W), lambda i: (0, 0, 0)),
            pl.BlockSpec((B * NC, 1, H), lambda i: (0, 0, 0)),
            pl.BlockSpec((B, 4, H * W), lambda i: (0, 0, 0)),
        ],
        out_specs=pl.BlockSpec((B, _K, 8), lambda i: (0, 0, 0)),
        out_shape=jax.ShapeDtypeStruct((B, _K, 8), jnp.float32),
        scratch_shapes=[pltpu.VMEM((B, NC), jnp.float32)],
    )(scores, rowmax, whreg)

    return det[:, :, :6]
